# baseline (speedup 1.0000x reference)
import jax
import jax.numpy as jnp
from jax.experimental import pallas as pl
from jax.experimental.pallas import tpu as pltpu

_TB = 4


def _body(img_ref, w_ref, p_ref, o_ref, m_ref):
    @pl.when(pl.program_id(0) == 0)
    def _():
        m_ref[...] = jnp.where(p_ref[...], 0.0, w_ref[...])

    o_ref[...] = img_ref[...] * m_ref[...][None, :, :]


def kernel(image, weight, pruned):
    B, C, H, W = image.shape
    BC = B * C
    img = image.reshape(BC, H, W)
    out = pl.pallas_call(
        _body,
        grid=(BC // _TB,),
        in_specs=[
            pl.BlockSpec((_TB, H, W), lambda i: (i, 0, 0)),
            pl.BlockSpec((H, W), lambda i: (0, 0)),
            pl.BlockSpec((H, W), lambda i: (0, 0)),
        ],
        out_specs=pl.BlockSpec((_TB, H, W), lambda i: (i, 0, 0)),
        out_shape=jax.ShapeDtypeStruct((BC, H, W), image.dtype),
        scratch_shapes=[pltpu.VMEM((H, W), jnp.float32)],
        compiler_params=pltpu.CompilerParams(
            dimension_semantics=("arbitrary",),
        ),
    )(img, weight, pruned)
    return out.reshape(1, B, C, H, W)


# speedup vs baseline: 1.0143x; 1.0143x over previous
import jax
import jax.numpy as jnp
from jax.experimental import pallas as pl
from jax.experimental.pallas import tpu as pltpu

_CB = 2
_NBUF = 6
_LEAD = 4


def _body(img_hbm, w_ref, p_ref, out_hbm, buf, m_ref, rsem, wsem):
    n = img_hbm.shape[0] // _CB

    m_ref[...] = jnp.where(p_ref[...], 0.0, w_ref[...])

    def read(j):
        pltpu.make_async_copy(
            img_hbm.at[pl.ds(j * _CB, _CB)], buf.at[j % _NBUF], rsem.at[j % _NBUF]
        ).start()

    def write_copy(i):
        return pltpu.make_async_copy(
            buf.at[i % _NBUF], out_hbm.at[pl.ds(i * _CB, _CB)], wsem.at[i % _NBUF]
        )

    for j in range(_LEAD):
        read(j)

    for i in range(n):
        j = i + _LEAD
        if j < n:
            if j - _NBUF >= 0:
                write_copy(j - _NBUF).wait()
            read(j)
        pltpu.make_async_copy(
            img_hbm.at[pl.ds(i * _CB, _CB)], buf.at[i % _NBUF], rsem.at[i % _NBUF]
        ).wait()
        buf[i % _NBUF] = buf[i % _NBUF] * m_ref[...][None, :, :]
        write_copy(i).start()

    for i in range(n - _NBUF, n):
        write_copy(i).wait()


def kernel(image, weight, pruned):
    B, C, H, W = image.shape
    BC = B * C
    img = image.reshape(BC, H, W)
    out = pl.pallas_call(
        _body,
        in_specs=[
            pl.BlockSpec(memory_space=pl.ANY),
            pl.BlockSpec((H, W), lambda: (0, 0)),
            pl.BlockSpec((H, W), lambda: (0, 0)),
        ],
        out_specs=pl.BlockSpec(memory_space=pl.ANY),
        out_shape=jax.ShapeDtypeStruct((BC, H, W), image.dtype),
        scratch_shapes=[
            pltpu.VMEM((_NBUF, _CB, H, W), jnp.float32),
            pltpu.VMEM((H, W), jnp.float32),
            pltpu.SemaphoreType.DMA((_NBUF,)),
            pltpu.SemaphoreType.DMA((_NBUF,)),
        ],
    )(img, weight, pruned)
    return out.reshape(1, B, C, H, W)


# speedup vs baseline: 1.0149x; 1.0006x over previous
import jax
import jax.numpy as jnp
from jax.experimental import pallas as pl
from jax.experimental.pallas import tpu as pltpu

_CB = 4
_NBUF = 4
_LEAD = 2


def _body(img_hbm, w_ref, p_ref, out_hbm, buf, m_ref, rsem, wsem):
    n = img_hbm.shape[0] // _CB

    m_ref[...] = jnp.where(p_ref[...], 0.0, w_ref[...])

    def read(j):
        pltpu.make_async_copy(
            img_hbm.at[pl.ds(j * _CB, _CB)], buf.at[j % _NBUF], rsem.at[j % _NBUF]
        ).start()

    def write_copy(i):
        return pltpu.make_async_copy(
            buf.at[i % _NBUF], out_hbm.at[pl.ds(i * _CB, _CB)], wsem.at[i % _NBUF]
        )

    for j in range(_LEAD):
        read(j)

    for i in range(n):
        j = i + _LEAD
        if j < n:
            if j - _NBUF >= 0:
                write_copy(j - _NBUF).wait()
            read(j)
        pltpu.make_async_copy(
            img_hbm.at[pl.ds(i * _CB, _CB)], buf.at[i % _NBUF], rsem.at[i % _NBUF]
        ).wait()
        buf[i % _NBUF] = buf[i % _NBUF] * m_ref[...][None, :, :]
        write_copy(i).start()

    for i in range(n - _NBUF, n):
        write_copy(i).wait()


def kernel(image, weight, pruned):
    B, C, H, W = image.shape
    BC = B * C
    img = image.reshape(BC, H, W)
    out = pl.pallas_call(
        _body,
        in_specs=[
            pl.BlockSpec(memory_space=pl.ANY),
            pl.BlockSpec((H, W), lambda: (0, 0)),
            pl.BlockSpec((H, W), lambda: (0, 0)),
        ],
        out_specs=pl.BlockSpec(memory_space=pl.ANY),
        out_shape=jax.ShapeDtypeStruct((BC, H, W), image.dtype),
        scratch_shapes=[
            pltpu.VMEM((_NBUF, _CB, H, W), jnp.float32),
            pltpu.VMEM((H, W), jnp.float32),
            pltpu.SemaphoreType.DMA((_NBUF,)),
            pltpu.SemaphoreType.DMA((_NBUF,)),
        ],
    )(img, weight, pruned)
    return out.reshape(1, B, C, H, W)
